# Initial kernel scaffold; baseline (speedup 1.0000x reference)
import jax
import jax.numpy as jnp
from jax.experimental import pallas as pl


def kernel(x, w, b):
    raise NotImplementedError("write your pallas kernel here")


# speedup vs baseline: 2.9526x; 2.9526x over previous
from functools import partial

import jax
import jax.numpy as jnp
from jax.experimental import pallas as pl
from jax.experimental.pallas import tpu as pltpu

_KSZ = 3
_HALO = 64


def _conv_body(x_ref, w_ref, b_ref, o_ref, xs_ref, *, cg, q, wp):
    for ki in range(_KSZ):
        for kj in range(_KSZ):
            t = ki * _KSZ + kj
            st = _HALO + (ki - 1) * wp + (kj - 1)
            xs_ref[t * cg:(t + 1) * cg, :] = x_ref[:, st:st + q]
    o_ref[...] = (
        jnp.dot(w_ref[...], xs_ref[...], preferred_element_type=jnp.float32)
        + b_ref[...]
    )


def kernel(x, w, b):
    B, C, H, W = x.shape
    G, N, K = w.shape
    cg = C // G
    hp, wp = H + 4, W + 2
    q = hp * wp
    xl = q + 2 * _HALO

    xp = jnp.pad(x, ((0, 0), (0, 0), (2, 2), (1, 1)))
    xf = jnp.pad(xp.reshape(B, C, q), ((0, 0), (0, 0), (_HALO, _HALO)))
    xf = xf.astype(jnp.bfloat16)

    wt = w.reshape(G, N, cg, _KSZ, _KSZ).transpose(0, 1, 3, 4, 2)
    wf = wt.reshape(G, N, _KSZ * _KSZ * cg).astype(jnp.bfloat16)
    bf = b.reshape(G, N, 1).astype(jnp.float32)

    body = partial(_conv_body, cg=cg, q=q, wp=wp)

    out = pl.pallas_call(
        body,
        out_shape=jax.ShapeDtypeStruct((B, G * N, q), jnp.float32),
        grid_spec=pltpu.PrefetchScalarGridSpec(
            num_scalar_prefetch=0,
            grid=(G, B),
            in_specs=[
                pl.BlockSpec((pl.Squeezed(), cg, xl), lambda g, bb: (bb, g, 0)),
                pl.BlockSpec((pl.Squeezed(), N, _KSZ * _KSZ * cg),
                             lambda g, bb: (g, 0, 0)),
                pl.BlockSpec((pl.Squeezed(), N, 1), lambda g, bb: (g, 0, 0)),
            ],
            out_specs=pl.BlockSpec((pl.Squeezed(), N, q),
                                   lambda g, bb: (bb, g, 0)),
            scratch_shapes=[pltpu.VMEM((_KSZ * _KSZ * cg, q), jnp.bfloat16)],
        ),
        compiler_params=pltpu.CompilerParams(
            dimension_semantics=("parallel", "parallel"),
            vmem_limit_bytes=64 * 1024 * 1024,
        ),
    )(xf, wf, bf)

    y = out.reshape(B, G * N, hp, wp)[:, :, 2:H + 2, 1:W + 1]
    y = y.reshape(B, G, N, H, W).transpose(0, 1, 3, 4, 2)
    return y.reshape(B, G * N, H, W)


# speedup vs baseline: 3.7449x; 1.2683x over previous
from functools import partial

import jax
import jax.numpy as jnp
from jax.experimental import pallas as pl
from jax.experimental.pallas import tpu as pltpu

_KSZ = 3
_HALO = 64


def _conv_body(x_ref, w_ref, b_ref, o_ref, xs_ref, acc_ref,
               *, cg, q, wp, h, wsp):
    for ki in range(_KSZ):
        for kj in range(_KSZ):
            t = ki * _KSZ + kj
            st = _HALO + (ki - 1) * wp + (kj - 1)
            xs_ref[t * cg:(t + 1) * cg, :] = x_ref[:, st:st + q]
    acc_ref[...] = jax.lax.dot_general(
        xs_ref[...], w_ref[...],
        dimension_numbers=(((0,), (0,)), ((), ())),
        preferred_element_type=jnp.float32,
    ) + b_ref[...]
    for i in range(h):
        base = (i + 2) * wp + 1
        o_ref[i * wsp:(i + 1) * wsp, :] = acc_ref[base:base + wsp, :]


def kernel(x, w, b):
    B, C, H, W = x.shape
    G, N, K = w.shape
    cg = C // G
    hp, wp = H + 4, W + 2
    q = hp * wp
    xl = q + 2 * _HALO

    xp = jnp.pad(x, ((0, 0), (0, 0), (2, 2), (1, 1)))
    xf = jnp.pad(xp.reshape(B, C, q), ((0, 0), (0, 0), (_HALO, _HALO)))
    xf = xf.astype(jnp.bfloat16)

    wt = w.reshape(G, N, cg, _KSZ, _KSZ).transpose(0, 3, 4, 2, 1)
    wf = wt.reshape(G, _KSZ * _KSZ * cg, N).astype(jnp.bfloat16)
    bf = b.reshape(G, 1, N).astype(jnp.float32)

    body = partial(_conv_body, cg=cg, q=q, wp=wp, h=H, wsp=W)

    out = pl.pallas_call(
        body,
        out_shape=jax.ShapeDtypeStruct((B, G, H * W, N), jnp.float32),
        grid_spec=pltpu.PrefetchScalarGridSpec(
            num_scalar_prefetch=0,
            grid=(G, B),
            in_specs=[
                pl.BlockSpec((pl.Squeezed(), cg, xl), lambda g, bb: (bb, g, 0)),
                pl.BlockSpec((pl.Squeezed(), _KSZ * _KSZ * cg, N),
                             lambda g, bb: (g, 0, 0)),
                pl.BlockSpec((pl.Squeezed(), 1, N), lambda g, bb: (g, 0, 0)),
            ],
            out_specs=pl.BlockSpec((pl.Squeezed(), pl.Squeezed(), H * W, N),
                                   lambda g, bb: (bb, g, 0, 0)),
            scratch_shapes=[
                pltpu.VMEM((_KSZ * _KSZ * cg, q), jnp.bfloat16),
                pltpu.VMEM((q, N), jnp.float32),
            ],
        ),
        compiler_params=pltpu.CompilerParams(
            dimension_semantics=("parallel", "parallel"),
            vmem_limit_bytes=48 * 1024 * 1024,
        ),
    )(xf, wf, bf)

    return out.reshape(B, G * N, H, W)


# speedup vs baseline: 4.2797x; 1.1428x over previous
from functools import partial

import jax
import jax.numpy as jnp
from jax.experimental import pallas as pl
from jax.experimental.pallas import tpu as pltpu

_KSZ = 3
_HALO = 64


def _conv_body(x_ref, w_ref, b_ref, o_ref, xs_ref, acc_ref,
               *, nb, cg, q, wp, h, wsp):
    for bi in range(nb):
        for ki in range(_KSZ):
            for kj in range(_KSZ):
                t = ki * _KSZ + kj
                st = _HALO + (ki - 1) * wp + (kj - 1)
                xs_ref[t * cg:(t + 1) * cg, :] = x_ref[bi, :, st:st + q]
        acc_ref[...] = jax.lax.dot_general(
            xs_ref[...], w_ref[...],
            dimension_numbers=(((0,), (0,)), ((), ())),
            preferred_element_type=jnp.float32,
        ) + b_ref[...]
        for i in range(h):
            base = (i + 2) * wp + 1
            o_ref[bi, i * wsp:(i + 1) * wsp, :] = acc_ref[base:base + wsp, :]


def kernel(x, w, b):
    B, C, H, W = x.shape
    G, N, K = w.shape
    cg = C // G
    hp, wp = H + 4, W + 2
    q = hp * wp
    xl = q + 2 * _HALO

    xp = jnp.pad(x, ((0, 0), (0, 0), (2, 2), (1, 1)))
    xf = jnp.pad(xp.reshape(B, C, q), ((0, 0), (0, 0), (_HALO, _HALO)))
    xf = xf.astype(jnp.bfloat16)

    wt = w.reshape(G, N, cg, _KSZ, _KSZ).transpose(0, 3, 4, 2, 1)
    wf = wt.reshape(G, _KSZ * _KSZ * cg, N).astype(jnp.bfloat16)
    bf = b.reshape(G, 1, N).astype(jnp.float32)

    nb = 8 if B % 8 == 0 else 1
    body = partial(_conv_body, nb=nb, cg=cg, q=q, wp=wp, h=H, wsp=W)

    out = pl.pallas_call(
        body,
        out_shape=jax.ShapeDtypeStruct((B, G, H * W, N), jnp.float32),
        grid_spec=pltpu.PrefetchScalarGridSpec(
            num_scalar_prefetch=0,
            grid=(G, B // nb),
            in_specs=[
                pl.BlockSpec((nb, cg, xl), lambda g, bb: (bb, g, 0)),
                pl.BlockSpec((pl.Squeezed(), _KSZ * _KSZ * cg, N),
                             lambda g, bb: (g, 0, 0)),
                pl.BlockSpec((pl.Squeezed(), 1, N), lambda g, bb: (g, 0, 0)),
            ],
            out_specs=pl.BlockSpec((nb, pl.Squeezed(), H * W, N),
                                   lambda g, bb: (bb, g, 0, 0)),
            scratch_shapes=[
                pltpu.VMEM((_KSZ * _KSZ * cg, q), jnp.bfloat16),
                pltpu.VMEM((q, N), jnp.float32),
            ],
        ),
        compiler_params=pltpu.CompilerParams(
            dimension_semantics=("parallel", "parallel"),
            vmem_limit_bytes=48 * 1024 * 1024,
        ),
    )(xf, wf, bf)

    return out.reshape(B, G * N, H, W)


# speedup vs baseline: 4.5870x; 1.0718x over previous
from functools import partial

import jax
import jax.numpy as jnp
from jax.experimental import pallas as pl
from jax.experimental.pallas import tpu as pltpu

_KSZ = 3
_HALO = 64


def _conv_body(x_ref, w_ref, b_ref, o_ref, xp_ref, xs_ref,
               *, nb, cg, l, wsp):
    j_idx = jax.lax.broadcasted_iota(jnp.int32, (1, l), 1) % wsp
    not_left, not_right = j_idx != 0, j_idx != wsp - 1

    xp_ref[:, 0:_HALO] = jnp.zeros((cg, _HALO), jnp.bfloat16)
    xp_ref[:, _HALO + l:] = jnp.zeros((cg, _HALO), jnp.bfloat16)

    for bi in range(nb):
        xp_ref[:, _HALO:_HALO + l] = x_ref[bi].astype(jnp.bfloat16)
        for ki in range(_KSZ):
            for kj in range(_KSZ):
                t = ki * _KSZ + kj
                st = _HALO + (ki - 1) * wsp + (kj - 1)
                val = xp_ref[:, st:st + l]
                if kj == 0:
                    val = jnp.where(not_left, val, jnp.bfloat16(0))
                elif kj == _KSZ - 1:
                    val = jnp.where(not_right, val, jnp.bfloat16(0))
                xs_ref[t * cg:(t + 1) * cg, :] = val
        o_ref[bi] = jax.lax.dot_general(
            xs_ref[...], w_ref[...],
            dimension_numbers=(((0,), (0,)), ((), ())),
            preferred_element_type=jnp.float32,
        ) + b_ref[...]


def kernel(x, w, b):
    B, C, H, W = x.shape
    G, N, K = w.shape
    cg = C // G
    l = H * W

    xf = x.reshape(B, C, l)

    wt = w.reshape(G, N, cg, _KSZ, _KSZ).transpose(0, 3, 4, 2, 1)
    wf = wt.reshape(G, _KSZ * _KSZ * cg, N).astype(jnp.bfloat16)
    bf = b.reshape(G, 1, N).astype(jnp.float32)

    nb = 8 if B % 8 == 0 else 1
    body = partial(_conv_body, nb=nb, cg=cg, l=l, wsp=W)

    out = pl.pallas_call(
        body,
        out_shape=jax.ShapeDtypeStruct((B, G, l, N), jnp.float32),
        grid_spec=pltpu.PrefetchScalarGridSpec(
            num_scalar_prefetch=0,
            grid=(G, B // nb),
            in_specs=[
                pl.BlockSpec((nb, cg, l), lambda g, bb: (bb, g, 0)),
                pl.BlockSpec((pl.Squeezed(), _KSZ * _KSZ * cg, N),
                             lambda g, bb: (g, 0, 0)),
                pl.BlockSpec((pl.Squeezed(), 1, N), lambda g, bb: (g, 0, 0)),
            ],
            out_specs=pl.BlockSpec((nb, pl.Squeezed(), l, N),
                                   lambda g, bb: (bb, g, 0, 0)),
            scratch_shapes=[
                pltpu.VMEM((cg, l + 2 * _HALO), jnp.bfloat16),
                pltpu.VMEM((_KSZ * _KSZ * cg, l), jnp.bfloat16),
            ],
        ),
        compiler_params=pltpu.CompilerParams(
            dimension_semantics=("parallel", "parallel"),
            vmem_limit_bytes=48 * 1024 * 1024,
        ),
    )(xf, wf, bf)

    return out.reshape(B, G * N, H, W)


# speedup vs baseline: 5.7623x; 1.2562x over previous
from functools import partial

import jax
import jax.numpy as jnp
from jax.experimental import pallas as pl
from jax.experimental.pallas import tpu as pltpu

_KSZ = 3
_HALO = 64


def _conv_body(x_ref, w_ref, b_ref, o_ref, xp_ref, xs_ref,
               *, nb, l, wsp, cp):
    li = jax.lax.broadcasted_iota(jnp.int32, (l, 1), 0) % wsp
    not_left, not_right = li != 0, li != wsp - 1

    xp_ref[0:_HALO, :] = jnp.zeros((_HALO, cp), jnp.bfloat16)
    xp_ref[_HALO + l:, :] = jnp.zeros((_HALO, cp), jnp.bfloat16)

    for bi in range(nb):
        xp_ref[_HALO:_HALO + l, :] = x_ref[bi].astype(jnp.bfloat16)
        for ki in range(_KSZ):
            for kj in range(_KSZ):
                t = ki * _KSZ + kj
                st = _HALO + (ki - 1) * wsp + (kj - 1)
                val = xp_ref[st:st + l, :]
                if kj == 0:
                    val = jnp.where(not_left, val, jnp.bfloat16(0))
                elif kj == _KSZ - 1:
                    val = jnp.where(not_right, val, jnp.bfloat16(0))
                xs_ref[:, t * cp:(t + 1) * cp] = val
        acc = jnp.dot(xs_ref[...], w_ref[...],
                      preferred_element_type=jnp.float32) + b_ref[...]
        n = acc.shape[1] // 2
        o_ref[bi, 0] = acc[:, 0:n]
        o_ref[bi, 1] = acc[:, n:]


def kernel(x, w, b):
    B, C, H, W = x.shape
    G, N, K = w.shape
    cg = C // G
    gc = G // 2
    cp = 2 * cg
    l = H * W

    xh = x.transpose(0, 2, 3, 1).reshape(B, l, C)

    wt = w.reshape(G, N, cg, _KSZ * _KSZ).transpose(0, 3, 2, 1)
    wt = wt.reshape(gc, 2, _KSZ * _KSZ, cg, N)
    eye = jnp.eye(2, dtype=wt.dtype)
    wp = jnp.einsum("chtkn,hj->cthkjn", wt, eye)
    wp = wp.reshape(gc, _KSZ * _KSZ * cp, 2 * N).astype(jnp.bfloat16)
    bp = b.reshape(gc, 1, 2 * N).astype(jnp.float32)

    nb = 8 if B % 8 == 0 else 1
    body = partial(_conv_body, nb=nb, l=l, wsp=W, cp=cp)

    out = pl.pallas_call(
        body,
        out_shape=jax.ShapeDtypeStruct((B, G, l, N), jnp.float32),
        grid_spec=pltpu.PrefetchScalarGridSpec(
            num_scalar_prefetch=0,
            grid=(gc, B // nb),
            in_specs=[
                pl.BlockSpec((nb, l, cp), lambda c, bb: (bb, 0, c)),
                pl.BlockSpec((pl.Squeezed(), _KSZ * _KSZ * cp, 2 * N),
                             lambda c, bb: (c, 0, 0)),
                pl.BlockSpec((pl.Squeezed(), 1, 2 * N),
                             lambda c, bb: (c, 0, 0)),
            ],
            out_specs=pl.BlockSpec((nb, 2, l, N),
                                   lambda c, bb: (bb, c, 0, 0)),
            scratch_shapes=[
                pltpu.VMEM((l + 2 * _HALO, cp), jnp.bfloat16),
                pltpu.VMEM((l, _KSZ * _KSZ * cp), jnp.bfloat16),
            ],
        ),
        compiler_params=pltpu.CompilerParams(
            dimension_semantics=("parallel", "parallel"),
            vmem_limit_bytes=48 * 1024 * 1024,
        ),
    )(xh, wp, bp)

    return out.reshape(B, G * N, H, W)


# speedup vs baseline: 17.6202x; 3.0578x over previous
from functools import partial

import jax
import jax.numpy as jnp
from jax.experimental import pallas as pl
from jax.experimental.pallas import tpu as pltpu

_KSZ = 3
_HALO = 64


def _conv_body(x_ref, w_ref, b_ref, o_ref, xp_ref, xs_ref, acc_ref,
               *, nb, l, wsp, cp):
    li = jax.lax.broadcasted_iota(jnp.int32, (l, 1), 0) % wsp
    not_left, not_right = li != 0, li != wsp - 1

    xp_ref[0:_HALO, :] = jnp.zeros((_HALO, cp), jnp.bfloat16)
    xp_ref[_HALO + l:, :] = jnp.zeros((_HALO, cp), jnp.bfloat16)

    for bi in range(nb):
        xp_ref[_HALO:_HALO + l, :] = x_ref[bi].astype(jnp.bfloat16)
        for ki in range(_KSZ):
            for kj in range(_KSZ):
                t = ki * _KSZ + kj
                st = _HALO + (ki - 1) * wsp + (kj - 1)
                val = xp_ref[st:st + l, :]
                if kj == 0:
                    val = jnp.where(not_left, val, jnp.bfloat16(0))
                elif kj == _KSZ - 1:
                    val = jnp.where(not_right, val, jnp.bfloat16(0))
                xs_ref[:, t * cp:(t + 1) * cp] = val
        acc = jnp.dot(xs_ref[...], w_ref[...],
                      preferred_element_type=jnp.float32) + b_ref[...]
        nn = acc.shape[1] // 2
        r = l // nn
        acc_ref[...] = acc.reshape(nn, r, 2 * nn)
        for jlo in range(r):
            for g2 in range(2):
                o_ref[bi, jlo * nn:(jlo + 1) * nn,
                      g2 * nn:(g2 + 1) * nn] = (
                    acc_ref[:, jlo, g2 * nn:(g2 + 1) * nn].T)


def kernel(x, w, b):
    B, C, H, W = x.shape
    G, N, K = w.shape
    cg = C // G
    gc = G // 2
    cp = 2 * cg
    l = H * W

    xh = x.transpose(0, 2, 3, 1).reshape(B, l, C)

    wt = w.reshape(G, N, cg, _KSZ * _KSZ).transpose(0, 3, 2, 1)
    wt = wt.reshape(gc, 2, _KSZ * _KSZ, cg, N)
    eye = jnp.eye(2, dtype=wt.dtype)
    wp = jnp.einsum("chtkn,hj->cthkjn", wt, eye)
    wp = wp.reshape(gc, _KSZ * _KSZ * cp, 2 * N).astype(jnp.bfloat16)
    bp = b.reshape(gc, 1, 2 * N).astype(jnp.float32)

    nb = 8 if B % 8 == 0 else 1
    body = partial(_conv_body, nb=nb, l=l, wsp=W, cp=cp)

    out = pl.pallas_call(
        body,
        out_shape=jax.ShapeDtypeStruct((B, l, G * N), jnp.float32),
        grid_spec=pltpu.PrefetchScalarGridSpec(
            num_scalar_prefetch=0,
            grid=(gc, B // nb),
            in_specs=[
                pl.BlockSpec((nb, l, cp), lambda c, bb: (bb, 0, c)),
                pl.BlockSpec((pl.Squeezed(), _KSZ * _KSZ * cp, 2 * N),
                             lambda c, bb: (c, 0, 0)),
                pl.BlockSpec((pl.Squeezed(), 1, 2 * N),
                             lambda c, bb: (c, 0, 0)),
            ],
            out_specs=pl.BlockSpec((nb, l, 2 * N),
                                   lambda c, bb: (bb, 0, c)),
            scratch_shapes=[
                pltpu.VMEM((l + 2 * _HALO, cp), jnp.bfloat16),
                pltpu.VMEM((l, _KSZ * _KSZ * cp), jnp.bfloat16),
                pltpu.VMEM((N, l // N, 2 * N), jnp.float32),
            ],
        ),
        compiler_params=pltpu.CompilerParams(
            dimension_semantics=("parallel", "parallel"),
            vmem_limit_bytes=48 * 1024 * 1024,
        ),
    )(xh, wp, bp)

    return out.reshape(B, H, W, G * N).transpose(0, 3, 1, 2)
